# Initial kernel scaffold; baseline (speedup 1.0000x reference)
import jax
import jax.numpy as jnp
from jax.experimental import pallas as pl


def kernel(x, table):
    raise NotImplementedError("write your pallas kernel here")


# speedup vs baseline: 1.5601x; 1.5601x over previous
import functools

import jax
import jax.numpy as jnp
from jax import lax
from jax.experimental import pallas as pl
from jax.experimental.pallas import tpu as pltpu
from jax.experimental.pallas import tpu_sc as plsc

_NUM_CORES = 2
_NUM_SUBCORES = 16
_NUM_WORKERS = _NUM_CORES * _NUM_SUBCORES


def _emb_lookup(flat_idx, table, *, chunk, n_chunks):
    b_per_w = chunk * n_chunks
    total = b_per_w * _NUM_WORKERS
    d = table.shape[1]
    mesh = plsc.VectorSubcoreMesh(core_axis_name="c", subcore_axis_name="s")

    @functools.partial(
        pl.kernel,
        out_type=jax.ShapeDtypeStruct((total, d), table.dtype),
        mesh=mesh,
        compiler_params=pltpu.CompilerParams(use_tc_tiling_on_sc=False),
        scratch_types=[
            pltpu.VMEM((chunk,), jnp.int32),
            pltpu.VMEM((chunk, d), table.dtype),
            pltpu.SemaphoreType.DMA,
        ],
    )
    def emb(idx_hbm, table_hbm, out_hbm, idx_v, rows_v, sem):
        wid = lax.axis_index("s") * _NUM_CORES + lax.axis_index("c")
        base = wid * b_per_w

        def body(c, _):
            off = base + c * chunk
            pltpu.sync_copy(idx_hbm.at[pl.ds(off, chunk)], idx_v)
            pltpu.async_copy(table_hbm.at[idx_v], rows_v, sem).wait()
            pltpu.sync_copy(rows_v, out_hbm.at[pl.ds(off, chunk)])
            return ()

        lax.fori_loop(0, n_chunks, body, (), unroll=False)

    return emb(flat_idx, table)


def kernel(x, table):
    b, f = x.shape
    v, d = table.shape
    total = b * f
    assert total % _NUM_WORKERS == 0
    b_per_w = total // _NUM_WORKERS
    n_chunks = 8
    assert b_per_w % n_chunks == 0
    chunk = b_per_w // n_chunks

    flat = x.reshape(total).astype(jnp.int32)
    out = _emb_lookup(flat, table, chunk=chunk, n_chunks=n_chunks)
    return out.reshape(b, f, d)


# speedup vs baseline: 1.5775x; 1.0112x over previous
import functools

import jax
import jax.numpy as jnp
from jax import lax
from jax.experimental import pallas as pl
from jax.experimental.pallas import tpu as pltpu
from jax.experimental.pallas import tpu_sc as plsc

_NUM_CORES = 2
_NUM_SUBCORES = 16
_NUM_WORKERS = _NUM_CORES * _NUM_SUBCORES


def _emb_lookup(flat_idx, table, *, chunk, n_chunks):
    b_per_w = chunk * n_chunks
    total = b_per_w * _NUM_WORKERS
    d = table.shape[1]
    mesh = plsc.VectorSubcoreMesh(core_axis_name="c", subcore_axis_name="s")

    @functools.partial(
        pl.kernel,
        out_type=jax.ShapeDtypeStruct((total, d), table.dtype),
        mesh=mesh,
        compiler_params=pltpu.CompilerParams(use_tc_tiling_on_sc=False),
        scratch_types=[
            pltpu.VMEM((chunk,), jnp.int32),
            pltpu.VMEM((chunk,), jnp.int32),
            pltpu.VMEM((chunk, d), table.dtype),
            pltpu.VMEM((chunk, d), table.dtype),
            pltpu.SemaphoreType.DMA,
            pltpu.SemaphoreType.DMA,
            pltpu.SemaphoreType.DMA,
            pltpu.SemaphoreType.DMA,
            pltpu.SemaphoreType.DMA,
            pltpu.SemaphoreType.DMA,
        ],
    )
    def emb(idx_hbm, table_hbm, out_hbm, i0, i1, r0, r1,
            si0, si1, sg0, sg1, ss0, ss1):
        wid = lax.axis_index("s") * _NUM_CORES + lax.axis_index("c")
        base = wid * b_per_w
        idx_bufs, row_bufs = [i0, i1], [r0, r1]
        sem_i, sem_g, sem_s = [si0, si1], [sg0, sg1], [ss0, ss1]

        def idx_load(c):
            b = c % 2
            return pltpu.make_async_copy(
                idx_hbm.at[pl.ds(base + c * chunk, chunk)], idx_bufs[b],
                sem_i[b])

        def gather(c):
            b = c % 2
            return pltpu.make_async_copy(
                table_hbm.at[idx_bufs[b]], row_bufs[b], sem_g[b])

        def store(c):
            b = c % 2
            return pltpu.make_async_copy(
                row_bufs[b], out_hbm.at[pl.ds(base + c * chunk, chunk)],
                sem_s[b])

        idx_load(0).start()
        if n_chunks > 1:
            idx_load(1).start()
        idx_load(0).wait()
        gather(0).start()

        for c in range(n_chunks):
            if c + 1 < n_chunks:
                if c >= 1:
                    store(c - 1).wait()
                idx_load(c + 1).wait()
                gather(c + 1).start()
            gather(c).wait()
            store(c).start()
            if c + 2 < n_chunks:
                idx_load(c + 2).start()

        if n_chunks > 1:
            store(n_chunks - 2).wait()
        store(n_chunks - 1).wait()

    return emb(flat_idx, table)


def kernel(x, table):
    b, f = x.shape
    v, d = table.shape
    total = b * f
    assert total % _NUM_WORKERS == 0
    b_per_w = total // _NUM_WORKERS
    n_chunks = 8
    assert b_per_w % n_chunks == 0
    chunk = b_per_w // n_chunks

    flat = x.reshape(total).astype(jnp.int32)
    out = _emb_lookup(flat, table, chunk=chunk, n_chunks=n_chunks)
    return out.reshape(b, f, d)
